# baseline (speedup 1.0000x reference)
import functools

import jax
import jax.numpy as jnp
from jax import lax
from jax.experimental import pallas as pl
from jax.experimental.pallas import tpu as pltpu
from jax.experimental.pallas import tpu_sc as plsc

N = 10000
E = 160000
IN_DIM = 256
HID = 512
NCLS = 40
L = 4

NP = 10240
R = 320
NC = 2
NS = 16
NT = NC * NS
EP = E + 2048
B = 128
HH = HID // 2
NCP = 48


def _al(i):
    return pl.multiple_of(i, 8)


def _mesh():
    return plsc.VectorSubcoreMesh(
        core_axis_name="c", subcore_axis_name="s", num_cores=NC, num_subcores=NS
    )


def _proj_body(x_ref, w_ref, b_ref, o0_ref, o1_ref):
    o = lax.dot_general(
        x_ref[...], w_ref[...], (((1,), (1,)), ((), ())),
        preferred_element_type=jnp.float32,
    ) + b_ref[...]
    o0_ref[...] = o[:, :HH]
    o1_ref[...] = o[:, HH:]


def _tc_in_proj(x, W, b):
    BM = 1024
    hspec = pl.BlockSpec((BM, HH), lambda i: (i, 0))
    hshp = jax.ShapeDtypeStruct((NP, HH), jnp.float32)
    return pl.pallas_call(
        _proj_body,
        grid=(NP // BM,),
        in_specs=[
            pl.BlockSpec((BM, IN_DIM), lambda i: (i, 0)),
            pl.BlockSpec((HID, IN_DIM), lambda i: (0, 0)),
            pl.BlockSpec((1, HID), lambda i: (0, 0)),
        ],
        out_specs=[hspec, hspec],
        out_shape=[hshp, hshp],
    )(x, W, b[None, :])


def _qkv_body(x0_ref, x1_ref, wq_ref, wk_ref, wv_ref, bq_ref, bk_ref, bv_ref,
              q0_ref, q1_ref, k0_ref, k1_ref, v0_ref, v1_ref):
    x0 = x0_ref[...]
    x1 = x1_ref[...]
    scaling = float(HID) ** -0.5

    def mm(w_ref, b_ref):
        dn = (((1,), (1,)), ((), ()))
        w = w_ref[...]
        return (lax.dot_general(x0, w[:, :HH], dn,
                                preferred_element_type=jnp.float32)
                + lax.dot_general(x1, w[:, HH:], dn,
                                  preferred_element_type=jnp.float32)
                + b_ref[...])

    q = mm(wq_ref, bq_ref) * scaling
    k = mm(wk_ref, bk_ref)
    v = mm(wv_ref, bv_ref)
    q0_ref[...] = q[:, :HH]
    q1_ref[...] = q[:, HH:]
    k0_ref[...] = k[:, :HH]
    k1_ref[...] = k[:, HH:]
    v0_ref[...] = v[:, :HH]
    v1_ref[...] = v[:, HH:]


def _tc_qkv(x0, x1, Wq, bq, Wk, bk, Wv, bv):
    BM = 1024
    wspec = pl.BlockSpec((HID, HID), lambda i: (0, 0))
    bspec = pl.BlockSpec((1, HID), lambda i: (0, 0))
    hspec = pl.BlockSpec((BM, HH), lambda i: (i, 0))
    hshp = jax.ShapeDtypeStruct((NP, HH), jnp.float32)
    return pl.pallas_call(
        _qkv_body,
        grid=(NP // BM,),
        in_specs=[hspec, hspec, wspec, wspec, wspec, bspec, bspec, bspec],
        out_specs=[hspec] * 6,
        out_shape=[hshp] * 6,
    )(x0, x1, Wq, Wk, Wv, bq[None, :], bk[None, :], bv[None, :])


def _out_body(x0_ref, x1_ref, w_ref, b_ref, o_ref):
    dn = (((1,), (1,)), ((), ()))
    w = w_ref[...]
    logits = (lax.dot_general(x0_ref[...], w[:, :HH], dn,
                              preferred_element_type=jnp.float32)
              + lax.dot_general(x1_ref[...], w[:, HH:], dn,
                                preferred_element_type=jnp.float32)
              + b_ref[...])
    col = lax.broadcasted_iota(jnp.int32, logits.shape, 1)
    z = jnp.where(col < NCLS, logits, -1e30)
    m = jnp.max(z, axis=1, keepdims=True)
    ez = jnp.exp(z - m)
    o_ref[...] = (z - m) - jnp.log(jnp.sum(ez, axis=1, keepdims=True))


def _tc_out(x0, x1, W, b):
    BM = 1024
    hspec = pl.BlockSpec((BM, HH), lambda i: (i, 0))
    return pl.pallas_call(
        _out_body,
        grid=(NP // BM,),
        in_specs=[
            hspec, hspec,
            pl.BlockSpec((NCP, HID), lambda i: (0, 0)),
            pl.BlockSpec((1, NCP), lambda i: (0, 0)),
        ],
        out_specs=pl.BlockSpec((BM, NCP), lambda i: (i, 0)),
        out_shape=jax.ShapeDtypeStruct((NP, NCP), jnp.float32),
    )(x0, x1, W, b[None, :])


G = 2000
STG = G + 160


def _bucket_body(src_hbm, dst_hbm, bsrc_hbm, bdstl_hbm, cnt_hbm,
                 dbuf, sbuf, stg_s, stg_d, cbuf, sem):
    c = lax.axis_index("c")
    s = lax.axis_index("s")
    w = c * NS + s
    start = w * R
    zero16 = jnp.zeros((16,), jnp.int32)

    def group(g, carry):
        wcnt, rem = carry
        pltpu.sync_copy(src_hbm.at[pl.ds(_al(g * G), G)], sbuf)
        pltpu.sync_copy(dst_hbm.at[pl.ds(_al(g * G), G)], dbuf)

        def step(t, cnt):
            sl = pl.ds(16 * t, 16)
            d16 = dbuf[sl]
            s16 = sbuf[sl]
            m = (d16 >= start) & (d16 < start + R)
            ps = plsc.cumsum(m.astype(jnp.int32))
            idx = cnt + ps - 1
            plsc.store_scatter(stg_s, [idx], s16, mask=m)
            plsc.store_scatter(stg_d, [idx], d16 - start, mask=m)
            return cnt + ps[15]

        cnt = lax.fori_loop(0, G // 16, step, rem)
        nch = cnt // 128

        def flush(i, _):
            pltpu.sync_copy(stg_s.at[pl.ds(128 * i, 128)],
                            bsrc_hbm.at[pl.ds(_al(w * EP + wcnt + 128 * i), 128)])
            pltpu.sync_copy(stg_d.at[pl.ds(128 * i, 128)],
                            bdstl_hbm.at[pl.ds(_al(w * EP + wcnt + 128 * i), 128)])
            return 0

        lax.fori_loop(0, nch, flush, 0)
        base = 128 * nch
        for i in range(8):
            stg_s[pl.ds(16 * i, 16)] = stg_s[pl.ds(base + 16 * i, 16)]
            stg_d[pl.ds(16 * i, 16)] = stg_d[pl.ds(base + 16 * i, 16)]
        return (wcnt + base, cnt - base)

    wcnt, rem = lax.fori_loop(0, E // G, group, (0, 0))
    for i in range(8):
        stg_s[pl.ds(rem + 16 * i, 16)] = zero16
        stg_d[pl.ds(rem + 16 * i, 16)] = zero16
    pltpu.sync_copy(stg_s.at[pl.ds(0, 128)],
                    bsrc_hbm.at[pl.ds(_al(w * EP + wcnt), 128)])
    pltpu.sync_copy(stg_d.at[pl.ds(0, 128)],
                    bdstl_hbm.at[pl.ds(_al(w * EP + wcnt), 128)])
    cw = wcnt + rem
    cbuf[...] = jnp.where(lax.iota(jnp.int32, 16) == 0, cw, 0)
    pltpu.sync_copy(cbuf, cnt_hbm.at[pl.ds(_al(w * 16), 16)])


def _sc_bucket(src, dst):
    ity = jnp.int32
    f = pl.kernel(
        _bucket_body,
        out_type=[
            jax.ShapeDtypeStruct((NT * EP,), ity),
            jax.ShapeDtypeStruct((NT * EP,), ity),
            jax.ShapeDtypeStruct((NT * 16,), ity),
        ],
        mesh=_mesh(),
        compiler_params=pltpu.CompilerParams(needs_layout_passes=False),
        scratch_types=[
            pltpu.VMEM((G,), ity),
            pltpu.VMEM((G,), ity),
            pltpu.VMEM((STG,), ity),
            pltpu.VMEM((STG,), ity),
            pltpu.VMEM((16,), ity),
            pltpu.SemaphoreType.DMA,
        ],
    )
    return f(src, dst)


SCAP = 8192


def _sort_body(bsrc_hbm, bdstl_hbm, cnt_hbm, osrc_hbm, odstl_hbm,
               ssrc, sdl, tsrc, tdl, cbuf, cnt, sem):
    i32 = jnp.int32
    c = lax.axis_index("c")
    s = lax.axis_index("s")
    w = c * NS + s
    ebase = w * EP
    pltpu.sync_copy(cnt_hbm.at[pl.ds(_al(w * 16), 16)], cbuf)
    cw = cbuf[...][0]
    nch = (cw + 127) // 128
    lane0 = lax.iota(i32, 16) == 0

    @pl.when((cw > 0) & (cw <= SCAP))
    def _():
        def load(i, _):
            pltpu.sync_copy(bsrc_hbm.at[pl.ds(_al(ebase + 128 * i), 128)],
                            ssrc.at[pl.ds(pl.multiple_of(128 * i, 8), 128)])
            pltpu.sync_copy(bdstl_hbm.at[pl.ds(_al(ebase + 128 * i), 128)],
                            sdl.at[pl.ds(pl.multiple_of(128 * i, 8), 128)])
            return 0

        lax.fori_loop(0, nch, load, 0)

        def zc(r, _):
            cnt[r] = 0
            return 0

        lax.fori_loop(0, R, zc, 0)

        def count(e, _):
            dl = sdl[pl.ds(e, 16)][0]
            cnt[dl] = cnt[dl] + 1
            return 0

        lax.fori_loop(0, cw, count, 0)

        def scan(r, run):
            t = cnt[r]
            cnt[r] = run
            return run + t

        lax.fori_loop(0, R, scan, 0)

        def scatter(e, _):
            dl = sdl[pl.ds(e, 16)][0]
            sv = ssrc[pl.ds(e, 16)][0]
            pos = cnt[dl]
            cnt[dl] = pos + 1
            pidx = jnp.full((16,), pos, i32)
            plsc.store_scatter(tsrc, [pidx], jnp.full((16,), sv, i32),
                               mask=lane0)
            plsc.store_scatter(tdl, [pidx], jnp.full((16,), dl, i32),
                               mask=lane0)
            return 0

        lax.fori_loop(0, cw, scatter, 0)
        zero16 = jnp.zeros((16,), i32)
        for i in range(8):
            tsrc[pl.ds(cw + 16 * i, 16)] = zero16
            tdl[pl.ds(cw + 16 * i, 16)] = zero16

        def store(i, _):
            pltpu.sync_copy(tsrc.at[pl.ds(pl.multiple_of(128 * i, 8), 128)],
                            osrc_hbm.at[pl.ds(_al(ebase + 128 * i), 128)])
            pltpu.sync_copy(tdl.at[pl.ds(pl.multiple_of(128 * i, 8), 128)],
                            odstl_hbm.at[pl.ds(_al(ebase + 128 * i), 128)])
            return 0

        lax.fori_loop(0, nch + 1, store, 0)

    @pl.when(cw > SCAP)
    def _():
        def copy(i, _):
            pltpu.sync_copy(bsrc_hbm.at[pl.ds(_al(ebase + 128 * i), 128)],
                            ssrc.at[pl.ds(0, 128)])
            pltpu.sync_copy(ssrc.at[pl.ds(0, 128)],
                            osrc_hbm.at[pl.ds(_al(ebase + 128 * i), 128)])
            pltpu.sync_copy(bdstl_hbm.at[pl.ds(_al(ebase + 128 * i), 128)],
                            sdl.at[pl.ds(0, 128)])
            pltpu.sync_copy(sdl.at[pl.ds(0, 128)],
                            odstl_hbm.at[pl.ds(_al(ebase + 128 * i), 128)])
            return 0

        lax.fori_loop(0, nch + 1, copy, 0)


def _sc_sort(bsrc, bdstl, cnts):
    ity = jnp.int32
    f = pl.kernel(
        _sort_body,
        out_type=[
            jax.ShapeDtypeStruct((NT * EP,), ity),
            jax.ShapeDtypeStruct((NT * EP,), ity),
        ],
        mesh=_mesh(),
        compiler_params=pltpu.CompilerParams(needs_layout_passes=False),
        scratch_types=[
            pltpu.VMEM((SCAP + 16,), ity),
            pltpu.VMEM((SCAP + 16,), ity),
            pltpu.VMEM((SCAP + 272,), ity),
            pltpu.VMEM((SCAP + 272,), ity),
            pltpu.VMEM((16,), ity),
            pltpu.SMEM((R,), ity),
            pltpu.SemaphoreType.DMA,
        ],
    )
    return f(bsrc, bdstl, cnts)


CH = 2048
BG = 64
NBG = CH // BG
NBLK = CH // 16


def _attn_body(q0, q1, k0, k1, v0, v1, bsrc, bdstl, cnts, xo0, xo1, scor,
               qbuf, ka, kb, esrc, edstl, escore, exb, smaxv, denomv, cbuf,
               smax, denom, sem):
    i32 = jnp.int32
    f32 = jnp.float32
    c = lax.axis_index("c")
    s = lax.axis_index("s")
    w = c * NS + s
    start = w * R
    ebase = w * EP
    pltpu.sync_copy(cnts.at[pl.ds(_al(w * 16), 16)], cbuf)
    cw = cbuf[...][0]
    nchunks = (cw + CH - 1) // CH

    def init_row(r, _):
        smax[r] = jnp.float32(-3.0e38)
        denom[r] = jnp.float32(0.0)
        return 0

    lax.fori_loop(0, R, init_row, 0)

    for hh in range(2):
        qh = (q0, q1)[hh]
        kh = (k0, k1)[hh]
        pltpu.sync_copy(qh.at[pl.ds(_al(start), R)], qbuf)
        qregs0 = tuple(qbuf[0, pl.ds(16 * j, 16)] for j in range(HH // 16))

        def sddmm_from(kbuf, boff, carry, hh=hh):
            for t in range(BG // 16):
                def edot(i, carry, t=t, boff=boff, kbuf=kbuf):
                    curq, qregs, vec = carry
                    ik = 16 * t + i
                    dl = edstl[pl.ds(boff + ik, 16)][0]

                    def reload(qr, dl=dl):
                        return tuple(qbuf[dl, pl.ds(16 * j, 16)]
                                     for j in range(HH // 16))

                    qregs = lax.cond(dl != curq, reload, lambda qr: qr, qregs)
                    acc = [jnp.zeros((16,), f32) for _ in range(4)]
                    for j in range(HH // 16):
                        slj = pl.ds(16 * j, 16)
                        acc[j % 4] = acc[j % 4] + qregs[j] * kbuf[ik, slj]
                    a = (acc[0] + acc[1]) + (acc[2] + acc[3])
                    vec = jnp.where(lax.iota(i32, 16) == i, jnp.sum(a), vec)
                    return (dl, qregs, vec)

                curq, qregs, vec = lax.fori_loop(
                    0, 16, edot, (carry[0], carry[1],
                                  jnp.zeros((16,), f32)))
                carry = (curq, qregs)
                pos = pl.ds(boff + 16 * t, 16)
                if hh:
                    vec = vec + escore[pos]
                escore[pos] = vec
            return carry

        def passA(ch, carry, hh=hh, kh=kh):
            base = _al(ebase + ch * CH)
            rem = cw - ch * CH
            npairs = jnp.minimum((rem + 2 * BG - 1) // (2 * BG), NBG // 2)
            pltpu.sync_copy(bsrc.at[pl.ds(base, CH)], esrc)
            pltpu.sync_copy(bdstl.at[pl.ds(base, CH)], edstl.at[pl.ds(0, CH)])
            if hh:
                pltpu.sync_copy(scor.at[pl.ds(base, CH)],
                                escore.at[pl.ds(0, CH)])
            pltpu.async_copy(kh.at[esrc.at[pl.ds(0, BG)]], ka, sem)

            def gbpair(p, carry, kh=kh, hh=hh):
                boff0 = 2 * p * BG
                boff1 = boff0 + BG
                pltpu.async_copy(kh.at[esrc.at[pl.ds(boff1, BG)]], kb, sem)
                pltpu.make_async_copy(kh.at[esrc.at[pl.ds(boff0, BG)]],
                                      ka, sem).wait()
                carry = sddmm_from(ka, boff0, carry)

                @pl.when(p + 1 < npairs)
                def _():
                    pltpu.async_copy(
                        kh.at[esrc.at[pl.ds(boff1 + BG, BG)]], ka, sem)

                pltpu.make_async_copy(kh.at[esrc.at[pl.ds(boff1, BG)]],
                                      kb, sem).wait()
                carry = sddmm_from(kb, boff1, carry)
                return carry

            carry = lax.fori_loop(0, npairs, gbpair, carry)
            pltpu.sync_copy(escore.at[pl.ds(0, CH)], scor.at[pl.ds(base, CH)])
            if hh:
                lim = jnp.minimum(CH, rem)

                def upd(e, _):
                    dl = edstl[pl.ds(e, 16)][0]
                    smax[dl] = jnp.maximum(smax[dl],
                                           escore[pl.ds(e, 16)][0])
                    return 0

                lax.fori_loop(0, lim, upd, 0)
            return carry

        lax.fori_loop(0, nchunks, passA, (0, qregs0))

    for t in range(R // 16):
        def fills(i, vs, t=t):
            r = 16 * t + i
            return jnp.where(lax.iota(i32, 16) == i, smax[r], vs)

        smaxv[pl.ds(16 * t, 16)] = lax.fori_loop(
            0, 16, fills, jnp.zeros((16,), f32))

    def passB(ch, _):
        base = _al(ebase + ch * CH)
        rem = cw - ch * CH
        pltpu.sync_copy(scor.at[pl.ds(base, CH)], escore.at[pl.ds(0, CH)])
        pltpu.sync_copy(bdstl.at[pl.ds(base, CH)], edstl.at[pl.ds(0, CH)])
        nblk = (jnp.minimum(CH, rem) + 15) // 16

        def blk(t, _):
            sl = pl.ds(16 * t, 16)
            dl16 = edstl[sl]
            eidx = lax.iota(i32, 16) + (ch * CH + 16 * t)
            dl16 = jnp.where(eidx < cw, dl16, 0)
            g = plsc.load_gather(smaxv, [dl16])
            exb[pl.ds(0, 16)] = jnp.exp(escore[sl] - g)
            limi = jnp.minimum(16, rem - 16 * t)

            def upd(i, _, t=t):
                dl = edstl[pl.ds(16 * t + i, 16)][0]
                denom[dl] = denom[dl] + exb[pl.ds(i, 16)][0]
                return 0

            lax.fori_loop(0, limi, upd, 0)
            return 0

        lax.fori_loop(0, nblk, blk, 0)
        return 0

    lax.fori_loop(0, nchunks, passB, 0)

    for t in range(R // 16):
        def filld(i, vd, t=t):
            r = 16 * t + i
            return jnp.where(lax.iota(i32, 16) == i, denom[r], vd)

        denomv[pl.ds(16 * t, 16)] = lax.fori_loop(
            0, 16, filld, jnp.zeros((16,), f32))

    for hh in range(2):
        vh = (v0, v1)[hh]

        def zrow(rr, _):
            for j in range(HH // 16):
                qbuf[rr, pl.ds(16 * j, 16)] = jnp.zeros((16,), f32)
            return 0

        lax.fori_loop(0, R, zrow, 0)

        def passC(ch, carry, hh=hh, vh=vh):
            base = _al(ebase + ch * CH)
            rem = cw - ch * CH
            npairs = jnp.minimum((rem + 2 * BG - 1) // (2 * BG), NBG // 2)
            pltpu.sync_copy(bsrc.at[pl.ds(base, CH)], esrc)
            pltpu.sync_copy(bdstl.at[pl.ds(base, CH)], edstl.at[pl.ds(0, CH)])
            pltpu.sync_copy(scor.at[pl.ds(base, CH)], escore.at[pl.ds(0, CH)])
            if hh == 0:
                def ablk(t, _):
                    sl = pl.ds(16 * t, 16)
                    dl16 = edstl[sl]
                    eidx = lax.iota(i32, 16) + (ch * CH + 16 * t)
                    valid = eidx < cw
                    dl16 = jnp.where(valid, dl16, 0)
                    g = plsc.load_gather(smaxv, [dl16])
                    ex = jnp.exp(escore[sl] - g)
                    dn = plsc.load_gather(denomv, [dl16])
                    escore[sl] = jnp.where(valid, ex / (dn + 1e-9), 0.0)
                    return 0

                lax.fori_loop(0, NBLK, ablk, 0)
                pltpu.sync_copy(escore.at[pl.ds(0, CH)],
                                scor.at[pl.ds(base, CH)])

            pltpu.async_copy(vh.at[esrc.at[pl.ds(0, BG)]], ka, sem)

            def accum_from(kbuf, boff, carry):
                def accum(i, carry, kbuf=kbuf, boff=boff):
                    cur, acc = carry
                    a = escore[pl.ds(boff + i, 16)][0]
                    dl = edstl[pl.ds(boff + i, 16)][0]
                    changed = dl != cur

                    def flushed(acc, cur=cur):
                        for j in range(HH // 16):
                            slj = pl.ds(16 * j, 16)
                            qbuf[cur, slj] = qbuf[cur, slj] + acc[j]
                        return tuple(jnp.zeros((16,), f32)
                                     for _ in range(HH // 16))

                    acc = lax.cond(changed, flushed, lambda a_: a_, acc)
                    newacc = []
                    for j in range(HH // 16):
                        slj = pl.ds(16 * j, 16)
                        newacc.append(acc[j] + kbuf[i, slj] * a)
                    return (dl, tuple(newacc))

                return lax.fori_loop(0, BG, accum, carry)

            def gbpair(p, carry, vh=vh):
                boff0 = 2 * p * BG
                boff1 = boff0 + BG
                pltpu.async_copy(vh.at[esrc.at[pl.ds(boff1, BG)]], kb, sem)
                pltpu.make_async_copy(vh.at[esrc.at[pl.ds(boff0, BG)]],
                                      ka, sem).wait()
                carry = accum_from(ka, boff0, carry)

                @pl.when(p + 1 < npairs)
                def _():
                    pltpu.async_copy(
                        vh.at[esrc.at[pl.ds(boff1 + BG, BG)]], ka, sem)

                pltpu.make_async_copy(vh.at[esrc.at[pl.ds(boff1, BG)]],
                                      kb, sem).wait()
                carry = accum_from(kb, boff1, carry)
                return carry

            return lax.fori_loop(0, npairs, gbpair, carry)

        zacc = tuple(jnp.zeros((16,), f32) for _ in range(HH // 16))
        cur, acc = lax.fori_loop(0, nchunks, passC, (0, zacc))
        for j in range(HH // 16):
            slj = pl.ds(16 * j, 16)
            qbuf[cur, slj] = qbuf[cur, slj] + acc[j]
        xoh = (xo0, xo1)[hh]
        pltpu.sync_copy(qbuf, xoh.at[pl.ds(_al(start), R)])


def _sc_attn(q0, q1, k0, k1, v0, v1, bsrc, bdstl, cnts):
    f32 = jnp.float32
    f = pl.kernel(
        _attn_body,
        out_type=[
            jax.ShapeDtypeStruct((NP, HH), f32),
            jax.ShapeDtypeStruct((NP, HH), f32),
            jax.ShapeDtypeStruct((NT * EP,), f32),
        ],
        mesh=_mesh(),
        compiler_params=pltpu.CompilerParams(needs_layout_passes=False),
        scratch_types=[
            pltpu.VMEM((R, HH), f32),
            pltpu.VMEM((BG, HH), f32),
            pltpu.VMEM((BG, HH), f32),
            pltpu.VMEM((CH,), jnp.int32),
            pltpu.VMEM((CH + 16,), jnp.int32),
            pltpu.VMEM((CH + 16,), f32),
            pltpu.VMEM((32,), f32),
            pltpu.VMEM((R,), f32),
            pltpu.VMEM((R,), f32),
            pltpu.VMEM((16,), jnp.int32),
            pltpu.SMEM((R,), f32),
            pltpu.SMEM((R,), f32),
            pltpu.SemaphoreType.DMA,
        ],
    )
    xo0, xo1, _ = f(q0, q1, k0, k1, v0, v1, bsrc, bdstl, cnts)
    return xo0, xo1


def kernel(h, in_W, in_b, qW, qb, kW, kb, vW, vb, out_W, out_b, edge_index):
    h_p = jnp.pad(h, ((0, NP - N), (0, 0)))
    x0, x1 = _tc_in_proj(h_p, in_W, in_b)
    src = edge_index[0]
    dst = edge_index[1]
    bsrc, bdstl, cnts = _sc_bucket(src, dst)
    bsrc, bdstl = _sc_sort(bsrc, bdstl, cnts)
    for l in range(L):
        q0, q1, k0, k1, v0, v1 = _tc_qkv(
            x0, x1, qW[l], qb[l], kW[l], kb[l], vW[l], vb[l])
        x0, x1 = _sc_attn(q0, q1, k0, k1, v0, v1, bsrc, bdstl, cnts)
    W_pad = jnp.pad(out_W, ((0, NCP - NCLS), (0, 0)))
    b_pad = jnp.pad(out_b, ((0, NCP - NCLS),))
    out = _tc_out(x0, x1, W_pad, b_pad)
    return out[:N, :NCLS]


# speedup vs baseline: 1.0378x; 1.0378x over previous
import jax
import jax.numpy as jnp
from jax import lax
from jax.experimental import pallas as pl
from jax.experimental.pallas import tpu as pltpu
from jax.experimental.pallas import tpu_sc as plsc

N = 10000
E = 160000
IN_DIM = 256
HID = 512
NCLS = 40
L = 4

NP = 10240
R = 320
NC = 2
NS = 16
NT = NC * NS
EP = E + 2048
B = 128
HH = HID // 2
NCP = 48


def _al(i):
    return pl.multiple_of(i, 8)


def _mesh():
    return plsc.VectorSubcoreMesh(
        core_axis_name="c", subcore_axis_name="s", num_cores=NC, num_subcores=NS
    )


def _proj_body(x_ref, w_ref, b_ref, o0_ref, o1_ref):
    o = lax.dot_general(
        x_ref[...], w_ref[...], (((1,), (1,)), ((), ())),
        preferred_element_type=jnp.float32,
    ) + b_ref[...]
    o0_ref[...] = o[:, :HH]
    o1_ref[...] = o[:, HH:]


def _tc_in_proj(x, W, b):
    BM = 1024
    hspec = pl.BlockSpec((BM, HH), lambda i: (i, 0))
    hshp = jax.ShapeDtypeStruct((NP, HH), jnp.float32)
    return pl.pallas_call(
        _proj_body,
        grid=(NP // BM,),
        in_specs=[
            pl.BlockSpec((BM, IN_DIM), lambda i: (i, 0)),
            pl.BlockSpec((HID, IN_DIM), lambda i: (0, 0)),
            pl.BlockSpec((1, HID), lambda i: (0, 0)),
        ],
        out_specs=[hspec, hspec],
        out_shape=[hshp, hshp],
    )(x, W, b[None, :])


def _qkv_body(x0_ref, x1_ref, wq_ref, wk_ref, wv_ref, bq_ref, bk_ref, bv_ref,
              q0_ref, q1_ref, k0_ref, k1_ref, v0_ref, v1_ref):
    x0 = x0_ref[...]
    x1 = x1_ref[...]
    scaling = float(HID) ** -0.5

    def mm(w_ref, b_ref):
        dn = (((1,), (1,)), ((), ()))
        w = w_ref[...]
        return (lax.dot_general(x0, w[:, :HH], dn,
                                preferred_element_type=jnp.float32)
                + lax.dot_general(x1, w[:, HH:], dn,
                                  preferred_element_type=jnp.float32)
                + b_ref[...])

    q = mm(wq_ref, bq_ref) * scaling
    k = mm(wk_ref, bk_ref)
    v = mm(wv_ref, bv_ref)
    q0_ref[...] = q[:, :HH]
    q1_ref[...] = q[:, HH:]
    k0_ref[...] = k[:, :HH]
    k1_ref[...] = k[:, HH:]
    v0_ref[...] = v[:, :HH]
    v1_ref[...] = v[:, HH:]


def _tc_qkv(x0, x1, Wq, bq, Wk, bk, Wv, bv):
    BM = 1024
    wspec = pl.BlockSpec((HID, HID), lambda i: (0, 0))
    bspec = pl.BlockSpec((1, HID), lambda i: (0, 0))
    hspec = pl.BlockSpec((BM, HH), lambda i: (i, 0))
    hshp = jax.ShapeDtypeStruct((NP, HH), jnp.float32)
    return pl.pallas_call(
        _qkv_body,
        grid=(NP // BM,),
        in_specs=[hspec, hspec, wspec, wspec, wspec, bspec, bspec, bspec],
        out_specs=[hspec] * 6,
        out_shape=[hshp] * 6,
    )(x0, x1, Wq, Wk, Wv, bq[None, :], bk[None, :], bv[None, :])


def _out_body(x0_ref, x1_ref, w_ref, b_ref, o_ref):
    dn = (((1,), (1,)), ((), ()))
    w = w_ref[...]
    logits = (lax.dot_general(x0_ref[...], w[:, :HH], dn,
                              preferred_element_type=jnp.float32)
              + lax.dot_general(x1_ref[...], w[:, HH:], dn,
                                preferred_element_type=jnp.float32)
              + b_ref[...])
    col = lax.broadcasted_iota(jnp.int32, logits.shape, 1)
    z = jnp.where(col < NCLS, logits, -1e30)
    m = jnp.max(z, axis=1, keepdims=True)
    ez = jnp.exp(z - m)
    o_ref[...] = (z - m) - jnp.log(jnp.sum(ez, axis=1, keepdims=True))


def _tc_out(x0, x1, W, b):
    BM = 1024
    hspec = pl.BlockSpec((BM, HH), lambda i: (i, 0))
    return pl.pallas_call(
        _out_body,
        grid=(NP // BM,),
        in_specs=[
            hspec, hspec,
            pl.BlockSpec((NCP, HID), lambda i: (0, 0)),
            pl.BlockSpec((1, NCP), lambda i: (0, 0)),
        ],
        out_specs=pl.BlockSpec((BM, NCP), lambda i: (i, 0)),
        out_shape=jax.ShapeDtypeStruct((NP, NCP), jnp.float32),
    )(x0, x1, W, b[None, :])


G = 2000
STG = G + 160


def _bucket_body(src_hbm, dst_hbm, bsrc_hbm, bdstl_hbm, cnt_hbm,
                 dbuf, sbuf, stg_s, stg_d, cbuf, sem):
    c = lax.axis_index("c")
    s = lax.axis_index("s")
    w = c * NS + s
    start = w * R
    zero16 = jnp.zeros((16,), jnp.int32)

    def group(g, carry):
        wcnt, rem = carry
        pltpu.sync_copy(src_hbm.at[pl.ds(_al(g * G), G)], sbuf)
        pltpu.sync_copy(dst_hbm.at[pl.ds(_al(g * G), G)], dbuf)

        def step(t, cnt):
            sl = pl.ds(16 * t, 16)
            d16 = dbuf[sl]
            s16 = sbuf[sl]
            m = (d16 >= start) & (d16 < start + R)
            ps = plsc.cumsum(m.astype(jnp.int32))
            idx = cnt + ps - 1
            plsc.store_scatter(stg_s, [idx], s16, mask=m)
            plsc.store_scatter(stg_d, [idx], d16 - start, mask=m)
            return cnt + ps[15]

        cnt = lax.fori_loop(0, G // 16, step, rem)
        nch = cnt // 128

        def flush(i, _):
            pltpu.sync_copy(stg_s.at[pl.ds(128 * i, 128)],
                            bsrc_hbm.at[pl.ds(_al(w * EP + wcnt + 128 * i), 128)])
            pltpu.sync_copy(stg_d.at[pl.ds(128 * i, 128)],
                            bdstl_hbm.at[pl.ds(_al(w * EP + wcnt + 128 * i), 128)])
            return 0

        lax.fori_loop(0, nch, flush, 0)
        base = 128 * nch
        for i in range(8):
            stg_s[pl.ds(16 * i, 16)] = stg_s[pl.ds(base + 16 * i, 16)]
            stg_d[pl.ds(16 * i, 16)] = stg_d[pl.ds(base + 16 * i, 16)]
        return (wcnt + base, cnt - base)

    wcnt, rem = lax.fori_loop(0, E // G, group, (0, 0))
    for i in range(8):
        stg_s[pl.ds(rem + 16 * i, 16)] = zero16
        stg_d[pl.ds(rem + 16 * i, 16)] = zero16
    pltpu.sync_copy(stg_s.at[pl.ds(0, 128)],
                    bsrc_hbm.at[pl.ds(_al(w * EP + wcnt), 128)])
    pltpu.sync_copy(stg_d.at[pl.ds(0, 128)],
                    bdstl_hbm.at[pl.ds(_al(w * EP + wcnt), 128)])
    cw = wcnt + rem
    cbuf[...] = jnp.where(lax.iota(jnp.int32, 16) == 0, cw, 0)
    pltpu.sync_copy(cbuf, cnt_hbm.at[pl.ds(_al(w * 16), 16)])


def _sc_bucket(src, dst):
    ity = jnp.int32
    f = pl.kernel(
        _bucket_body,
        out_type=[
            jax.ShapeDtypeStruct((NT * EP,), ity),
            jax.ShapeDtypeStruct((NT * EP,), ity),
            jax.ShapeDtypeStruct((NT * 16,), ity),
        ],
        mesh=_mesh(),
        compiler_params=pltpu.CompilerParams(needs_layout_passes=False),
        scratch_types=[
            pltpu.VMEM((G,), ity),
            pltpu.VMEM((G,), ity),
            pltpu.VMEM((STG,), ity),
            pltpu.VMEM((STG,), ity),
            pltpu.VMEM((16,), ity),
            pltpu.SemaphoreType.DMA,
        ],
    )
    return f(src, dst)


SCAP = 8192


def _sort_body(bsrc_hbm, bdstl_hbm, cnt_hbm, osrc_hbm, odstl_hbm,
               ssrc, sdl, tsrc, tdl, cbuf, cnt, sem):
    i32 = jnp.int32
    c = lax.axis_index("c")
    s = lax.axis_index("s")
    w = c * NS + s
    ebase = w * EP
    pltpu.sync_copy(cnt_hbm.at[pl.ds(_al(w * 16), 16)], cbuf)
    cw = cbuf[...][0]
    nch = (cw + 127) // 128
    lane0 = lax.iota(i32, 16) == 0

    @pl.when((cw > 0) & (cw <= SCAP))
    def _():
        def load(i, _):
            pltpu.sync_copy(bsrc_hbm.at[pl.ds(_al(ebase + 128 * i), 128)],
                            ssrc.at[pl.ds(pl.multiple_of(128 * i, 8), 128)])
            pltpu.sync_copy(bdstl_hbm.at[pl.ds(_al(ebase + 128 * i), 128)],
                            sdl.at[pl.ds(pl.multiple_of(128 * i, 8), 128)])
            return 0

        lax.fori_loop(0, nch, load, 0)

        def zc(r, _):
            cnt[r] = 0
            return 0

        lax.fori_loop(0, R, zc, 0)

        def count(e, _):
            dl = sdl[pl.ds(e, 16)][0]
            cnt[dl] = cnt[dl] + 1
            return 0

        lax.fori_loop(0, cw, count, 0)

        def scan(r, run):
            t = cnt[r]
            cnt[r] = run
            return run + t

        lax.fori_loop(0, R, scan, 0)

        def scatter(e, _):
            dl = sdl[pl.ds(e, 16)][0]
            sv = ssrc[pl.ds(e, 16)][0]
            pos = cnt[dl]
            cnt[dl] = pos + 1
            pidx = jnp.full((16,), pos, i32)
            plsc.store_scatter(tsrc, [pidx], jnp.full((16,), sv, i32),
                               mask=lane0)
            plsc.store_scatter(tdl, [pidx], jnp.full((16,), dl, i32),
                               mask=lane0)
            return 0

        lax.fori_loop(0, cw, scatter, 0)
        zero16 = jnp.zeros((16,), i32)
        for i in range(8):
            tsrc[pl.ds(cw + 16 * i, 16)] = zero16
            tdl[pl.ds(cw + 16 * i, 16)] = zero16

        def store(i, _):
            pltpu.sync_copy(tsrc.at[pl.ds(pl.multiple_of(128 * i, 8), 128)],
                            osrc_hbm.at[pl.ds(_al(ebase + 128 * i), 128)])
            pltpu.sync_copy(tdl.at[pl.ds(pl.multiple_of(128 * i, 8), 128)],
                            odstl_hbm.at[pl.ds(_al(ebase + 128 * i), 128)])
            return 0

        lax.fori_loop(0, nch + 1, store, 0)

    @pl.when(cw > SCAP)
    def _():
        def copy(i, _):
            pltpu.sync_copy(bsrc_hbm.at[pl.ds(_al(ebase + 128 * i), 128)],
                            ssrc.at[pl.ds(0, 128)])
            pltpu.sync_copy(ssrc.at[pl.ds(0, 128)],
                            osrc_hbm.at[pl.ds(_al(ebase + 128 * i), 128)])
            pltpu.sync_copy(bdstl_hbm.at[pl.ds(_al(ebase + 128 * i), 128)],
                            sdl.at[pl.ds(0, 128)])
            pltpu.sync_copy(sdl.at[pl.ds(0, 128)],
                            odstl_hbm.at[pl.ds(_al(ebase + 128 * i), 128)])
            return 0

        lax.fori_loop(0, nch + 1, copy, 0)


def _sc_sort(bsrc, bdstl, cnts):
    ity = jnp.int32
    f = pl.kernel(
        _sort_body,
        out_type=[
            jax.ShapeDtypeStruct((NT * EP,), ity),
            jax.ShapeDtypeStruct((NT * EP,), ity),
        ],
        mesh=_mesh(),
        compiler_params=pltpu.CompilerParams(needs_layout_passes=False),
        scratch_types=[
            pltpu.VMEM((SCAP + 16,), ity),
            pltpu.VMEM((SCAP + 16,), ity),
            pltpu.VMEM((SCAP + 272,), ity),
            pltpu.VMEM((SCAP + 272,), ity),
            pltpu.VMEM((16,), ity),
            pltpu.SMEM((R,), ity),
            pltpu.SemaphoreType.DMA,
        ],
    )
    return f(bsrc, bdstl, cnts)


CH = 2048
BG = 64
NBG = CH // BG
NBLK = CH // 16


def _attn_body(q0, q1, k0, k1, v0, v1, bsrc, bdstl, cnts, xo0, xo1, scor,
               qbuf, ka, kb, esrc, edstl, escore, exb, smaxv, denomv, cbuf,
               smax, denom, sem):
    i32 = jnp.int32
    f32 = jnp.float32
    c = lax.axis_index("c")
    s = lax.axis_index("s")
    w = c * NS + s
    start = w * R
    ebase = w * EP
    pltpu.sync_copy(cnts.at[pl.ds(_al(w * 16), 16)], cbuf)
    cw = cbuf[...][0]
    nchunks = (cw + CH - 1) // CH

    def init_row(r, _):
        smax[r] = jnp.float32(-3.0e38)
        denom[r] = jnp.float32(0.0)
        return 0

    lax.fori_loop(0, R, init_row, 0)

    def passA(ch, _):
        base = _al(ebase + ch * CH)
        rem = cw - ch * CH
        nbg = jnp.minimum((rem + BG - 1) // BG, NBG)
        pltpu.sync_copy(bsrc.at[pl.ds(base, CH)], esrc)
        pltpu.sync_copy(bdstl.at[pl.ds(base, CH)], edstl.at[pl.ds(0, CH)])
        for hh in range(2):
            qh = (q0, q1)[hh]
            kh = (k0, k1)[hh]
            pltpu.sync_copy(qh.at[pl.ds(_al(start), R)], qbuf)

            pltpu.async_copy(kh.at[esrc.at[pl.ds(0, BG)]], ka, sem)

            def sddmm_from(kbuf, boff, hh=hh):
                for t in range(BG // 16):
                    def edot(i, vec, t=t, boff=boff, kbuf=kbuf):
                        ik = 16 * t + i
                        dl = edstl[pl.ds(boff + ik, 16)][0]
                        acc = [jnp.zeros((16,), f32) for _ in range(4)]
                        for j in range(HH // 16):
                            slj = pl.ds(16 * j, 16)
                            acc[j % 4] = acc[j % 4] + qbuf[dl, slj] * kbuf[ik, slj]
                        a = (acc[0] + acc[1]) + (acc[2] + acc[3])
                        return jnp.where(lax.iota(i32, 16) == i,
                                         jnp.sum(a), vec)

                    vec = lax.fori_loop(0, 16, edot, jnp.zeros((16,), f32))
                    pos = pl.ds(boff + 16 * t, 16)
                    if hh:
                        vec = vec + escore[pos]
                    escore[pos] = vec

            def gbatch(b, _, kh=kh, hh=hh):
                boff = b * BG

                @pl.when(b + 1 < nbg)
                def _():
                    noff = boff + BG

                    @pl.when((b + 1) % 2 == 0)
                    def _():
                        pltpu.async_copy(kh.at[esrc.at[pl.ds(noff, BG)]],
                                         ka, sem)

                    @pl.when((b + 1) % 2 == 1)
                    def _():
                        pltpu.async_copy(kh.at[esrc.at[pl.ds(noff, BG)]],
                                         kb, sem)

                @pl.when(b % 2 == 0)
                def _():
                    pltpu.make_async_copy(kh.at[esrc.at[pl.ds(boff, BG)]],
                                          ka, sem).wait()
                    sddmm_from(ka, boff)

                @pl.when(b % 2 == 1)
                def _():
                    pltpu.make_async_copy(kh.at[esrc.at[pl.ds(boff, BG)]],
                                          kb, sem).wait()
                    sddmm_from(kb, boff)

                return 0

            lax.fori_loop(0, nbg, gbatch, 0)

        lim = jnp.minimum(CH, rem)

        def upd(e, _):
            dl = edstl[pl.ds(e, 16)][0]
            smax[dl] = jnp.maximum(smax[dl], escore[pl.ds(e, 16)][0])
            return 0

        lax.fori_loop(0, lim, upd, 0)
        pltpu.sync_copy(escore.at[pl.ds(0, CH)], scor.at[pl.ds(base, CH)])
        return 0

    lax.fori_loop(0, nchunks, passA, 0)

    for t in range(R // 16):
        def fills(i, vs, t=t):
            r = 16 * t + i
            return jnp.where(lax.iota(i32, 16) == i, smax[r], vs)

        smaxv[pl.ds(16 * t, 16)] = lax.fori_loop(
            0, 16, fills, jnp.zeros((16,), f32))

    def passB(ch, _):
        base = _al(ebase + ch * CH)
        rem = cw - ch * CH
        pltpu.sync_copy(scor.at[pl.ds(base, CH)], escore.at[pl.ds(0, CH)])
        pltpu.sync_copy(bdstl.at[pl.ds(base, CH)], edstl.at[pl.ds(0, CH)])
        nblk = (jnp.minimum(CH, rem) + 15) // 16

        def blk(t, _):
            sl = pl.ds(16 * t, 16)
            dl16 = edstl[sl]
            eidx = lax.iota(i32, 16) + (ch * CH + 16 * t)
            dl16 = jnp.where(eidx < cw, dl16, 0)
            g = plsc.load_gather(smaxv, [dl16])
            exb[pl.ds(0, 16)] = jnp.exp(escore[sl] - g)
            limi = jnp.minimum(16, rem - 16 * t)

            def upd(i, _, t=t):
                dl = edstl[pl.ds(16 * t + i, 16)][0]
                denom[dl] = denom[dl] + exb[pl.ds(i, 16)][0]
                return 0

            lax.fori_loop(0, limi, upd, 0)
            return 0

        lax.fori_loop(0, nblk, blk, 0)
        return 0

    lax.fori_loop(0, nchunks, passB, 0)

    for t in range(R // 16):
        def filld(i, vd, t=t):
            r = 16 * t + i
            return jnp.where(lax.iota(i32, 16) == i, denom[r], vd)

        denomv[pl.ds(16 * t, 16)] = lax.fori_loop(
            0, 16, filld, jnp.zeros((16,), f32))

    for hh in range(2):
        vh = (v0, v1)[hh]

        def zrow(rr, _):
            for j in range(HH // 16):
                qbuf[rr, pl.ds(16 * j, 16)] = jnp.zeros((16,), f32)
            return 0

        lax.fori_loop(0, R, zrow, 0)

        def passC(ch, carry, hh=hh, vh=vh):
            base = _al(ebase + ch * CH)
            rem = cw - ch * CH
            npairs = jnp.minimum((rem + 2 * BG - 1) // (2 * BG), NBG // 2)
            pltpu.sync_copy(bsrc.at[pl.ds(base, CH)], esrc)
            pltpu.sync_copy(bdstl.at[pl.ds(base, CH)], edstl.at[pl.ds(0, CH)])
            pltpu.sync_copy(scor.at[pl.ds(base, CH)], escore.at[pl.ds(0, CH)])
            if hh == 0:
                def ablk(t, _):
                    sl = pl.ds(16 * t, 16)
                    dl16 = edstl[sl]
                    eidx = lax.iota(i32, 16) + (ch * CH + 16 * t)
                    valid = eidx < cw
                    dl16 = jnp.where(valid, dl16, 0)
                    g = plsc.load_gather(smaxv, [dl16])
                    ex = jnp.exp(escore[sl] - g)
                    dn = plsc.load_gather(denomv, [dl16])
                    escore[sl] = jnp.where(valid, ex / (dn + 1e-9), 0.0)
                    return 0

                lax.fori_loop(0, NBLK, ablk, 0)
                pltpu.sync_copy(escore.at[pl.ds(0, CH)],
                                scor.at[pl.ds(base, CH)])

            pltpu.async_copy(vh.at[esrc.at[pl.ds(0, BG)]], ka, sem)

            def accum_from(kbuf, boff, carry):
                def accum(i, carry, kbuf=kbuf, boff=boff):
                    cur, acc = carry
                    a = escore[pl.ds(boff + i, 16)][0]
                    dl = edstl[pl.ds(boff + i, 16)][0]
                    changed = dl != cur

                    def flushed(acc, cur=cur):
                        for j in range(HH // 16):
                            slj = pl.ds(16 * j, 16)
                            qbuf[cur, slj] = qbuf[cur, slj] + acc[j]
                        return tuple(jnp.zeros((16,), f32)
                                     for _ in range(HH // 16))

                    acc = lax.cond(changed, flushed, lambda a_: a_, acc)
                    newacc = []
                    for j in range(HH // 16):
                        slj = pl.ds(16 * j, 16)
                        newacc.append(acc[j] + kbuf[i, slj] * a)
                    return (dl, tuple(newacc))

                return lax.fori_loop(0, BG, accum, carry)

            def gbpair(p, carry, vh=vh):
                boff0 = 2 * p * BG
                boff1 = boff0 + BG
                pltpu.async_copy(vh.at[esrc.at[pl.ds(boff1, BG)]], kb, sem)
                pltpu.make_async_copy(vh.at[esrc.at[pl.ds(boff0, BG)]],
                                      ka, sem).wait()
                carry = accum_from(ka, boff0, carry)

                @pl.when(p + 1 < npairs)
                def _():
                    pltpu.async_copy(
                        vh.at[esrc.at[pl.ds(boff1 + BG, BG)]], ka, sem)

                pltpu.make_async_copy(vh.at[esrc.at[pl.ds(boff1, BG)]],
                                      kb, sem).wait()
                carry = accum_from(kb, boff1, carry)
                return carry

            return lax.fori_loop(0, npairs, gbpair, carry)

        zacc = tuple(jnp.zeros((16,), f32) for _ in range(HH // 16))
        cur, acc = lax.fori_loop(0, nchunks, passC, (0, zacc))
        for j in range(HH // 16):
            slj = pl.ds(16 * j, 16)
            qbuf[cur, slj] = qbuf[cur, slj] + acc[j]
        xoh = (xo0, xo1)[hh]
        pltpu.sync_copy(qbuf, xoh.at[pl.ds(_al(start), R)])


def _sc_attn(q0, q1, k0, k1, v0, v1, bsrc, bdstl, cnts):
    f32 = jnp.float32
    f = pl.kernel(
        _attn_body,
        out_type=[
            jax.ShapeDtypeStruct((NP, HH), f32),
            jax.ShapeDtypeStruct((NP, HH), f32),
            jax.ShapeDtypeStruct((NT * EP,), f32),
        ],
        mesh=_mesh(),
        compiler_params=pltpu.CompilerParams(needs_layout_passes=False),
        scratch_types=[
            pltpu.VMEM((R, HH), f32),
            pltpu.VMEM((BG, HH), f32),
            pltpu.VMEM((BG, HH), f32),
            pltpu.VMEM((CH,), jnp.int32),
            pltpu.VMEM((CH + 16,), jnp.int32),
            pltpu.VMEM((CH + 16,), f32),
            pltpu.VMEM((32,), f32),
            pltpu.VMEM((R,), f32),
            pltpu.VMEM((R,), f32),
            pltpu.VMEM((16,), jnp.int32),
            pltpu.SMEM((R,), f32),
            pltpu.SMEM((R,), f32),
            pltpu.SemaphoreType.DMA,
        ],
    )
    xo0, xo1, _ = f(q0, q1, k0, k1, v0, v1, bsrc, bdstl, cnts)
    return xo0, xo1


def kernel(h, in_W, in_b, qW, qb, kW, kb, vW, vb, out_W, out_b, edge_index):
    h_p = jnp.pad(h, ((0, NP - N), (0, 0)))
    x0, x1 = _tc_in_proj(h_p, in_W, in_b)
    src = edge_index[0]
    dst = edge_index[1]
    bsrc, bdstl, cnts = _sc_bucket(src, dst)
    bsrc, bdstl = _sc_sort(bsrc, bdstl, cnts)
    for l in range(L):
        q0, q1, k0, k1, v0, v1 = _tc_qkv(
            x0, x1, qW[l], qb[l], kW[l], kb[l], vW[l], vb[l])
        x0, x1 = _sc_attn(q0, q1, k0, k1, v0, v1, bsrc, bdstl, cnts)
    W_pad = jnp.pad(out_W, ((0, NCP - NCLS), (0, 0)))
    b_pad = jnp.pad(out_b, ((0, NCP - NCLS),))
    out = _tc_out(x0, x1, W_pad, b_pad)
    return out[:N, :NCLS]


# speedup vs baseline: 1.0534x; 1.0150x over previous
import jax
import jax.numpy as jnp
from jax import lax
from jax.experimental import pallas as pl
from jax.experimental.pallas import tpu as pltpu
from jax.experimental.pallas import tpu_sc as plsc

N = 10000
E = 160000
IN_DIM = 256
HID = 512
NCLS = 40
L = 4

NP = 10240
R = 320
NC = 2
NS = 16
NT = NC * NS
EP = E + 4096
B = 128
HH = HID // 2
NCP = 48


def _al(i):
    return pl.multiple_of(i, 8)


def _mesh():
    return plsc.VectorSubcoreMesh(
        core_axis_name="c", subcore_axis_name="s", num_cores=NC, num_subcores=NS
    )


def _proj_body(x_ref, w_ref, b_ref, o0_ref, o1_ref):
    o = lax.dot_general(
        x_ref[...], w_ref[...], (((1,), (1,)), ((), ())),
        preferred_element_type=jnp.float32,
    ) + b_ref[...]
    o0_ref[...] = o[:, :HH]
    o1_ref[...] = o[:, HH:]


def _tc_in_proj(x, W, b):
    BM = 1024
    hspec = pl.BlockSpec((BM, HH), lambda i: (i, 0))
    hshp = jax.ShapeDtypeStruct((NP, HH), jnp.float32)
    return pl.pallas_call(
        _proj_body,
        grid=(NP // BM,),
        in_specs=[
            pl.BlockSpec((BM, IN_DIM), lambda i: (i, 0)),
            pl.BlockSpec((HID, IN_DIM), lambda i: (0, 0)),
            pl.BlockSpec((1, HID), lambda i: (0, 0)),
        ],
        out_specs=[hspec, hspec],
        out_shape=[hshp, hshp],
    )(x, W, b[None, :])


def _qkv_body(x0_ref, x1_ref, wq_ref, wk_ref, wv_ref, bq_ref, bk_ref, bv_ref,
              q0_ref, q1_ref, k0_ref, k1_ref, v0_ref, v1_ref):
    x0 = x0_ref[...]
    x1 = x1_ref[...]
    scaling = float(HID) ** -0.5

    def mm(w_ref, b_ref):
        dn = (((1,), (1,)), ((), ()))
        w = w_ref[...]
        return (lax.dot_general(x0, w[:, :HH], dn,
                                preferred_element_type=jnp.float32)
                + lax.dot_general(x1, w[:, HH:], dn,
                                  preferred_element_type=jnp.float32)
                + b_ref[...])

    q = mm(wq_ref, bq_ref) * scaling
    k = mm(wk_ref, bk_ref)
    v = mm(wv_ref, bv_ref)
    q0_ref[...] = q[:, :HH]
    q1_ref[...] = q[:, HH:]
    k0_ref[...] = k[:, :HH]
    k1_ref[...] = k[:, HH:]
    v0_ref[...] = v[:, :HH]
    v1_ref[...] = v[:, HH:]


def _tc_qkv(x0, x1, Wq, bq, Wk, bk, Wv, bv):
    BM = 1024
    wspec = pl.BlockSpec((HID, HID), lambda i: (0, 0))
    bspec = pl.BlockSpec((1, HID), lambda i: (0, 0))
    hspec = pl.BlockSpec((BM, HH), lambda i: (i, 0))
    hshp = jax.ShapeDtypeStruct((NP, HH), jnp.float32)
    return pl.pallas_call(
        _qkv_body,
        grid=(NP // BM,),
        in_specs=[hspec, hspec, wspec, wspec, wspec, bspec, bspec, bspec],
        out_specs=[hspec] * 6,
        out_shape=[hshp] * 6,
    )(x0, x1, Wq, Wk, Wv, bq[None, :], bk[None, :], bv[None, :])


def _out_body(x0_ref, x1_ref, w_ref, b_ref, o_ref):
    dn = (((1,), (1,)), ((), ()))
    w = w_ref[...]
    logits = (lax.dot_general(x0_ref[...], w[:, :HH], dn,
                              preferred_element_type=jnp.float32)
              + lax.dot_general(x1_ref[...], w[:, HH:], dn,
                                preferred_element_type=jnp.float32)
              + b_ref[...])
    col = lax.broadcasted_iota(jnp.int32, logits.shape, 1)
    z = jnp.where(col < NCLS, logits, -1e30)
    m = jnp.max(z, axis=1, keepdims=True)
    ez = jnp.exp(z - m)
    o_ref[...] = (z - m) - jnp.log(jnp.sum(ez, axis=1, keepdims=True))


def _tc_out(x0, x1, W, b):
    BM = 1024
    hspec = pl.BlockSpec((BM, HH), lambda i: (i, 0))
    return pl.pallas_call(
        _out_body,
        grid=(NP // BM,),
        in_specs=[
            hspec, hspec,
            pl.BlockSpec((NCP, HID), lambda i: (0, 0)),
            pl.BlockSpec((1, NCP), lambda i: (0, 0)),
        ],
        out_specs=pl.BlockSpec((BM, NCP), lambda i: (i, 0)),
        out_shape=jax.ShapeDtypeStruct((NP, NCP), jnp.float32),
    )(x0, x1, W, b[None, :])


G = 2000
STG = G + 160


def _bucket_body(src_hbm, dst_hbm, bsrc_hbm, bdstl_hbm, cnt_hbm,
                 dbuf, sbuf, stg_s, stg_d, cbuf, sem):
    c = lax.axis_index("c")
    s = lax.axis_index("s")
    w = c * NS + s
    start = w * R
    zero16 = jnp.zeros((16,), jnp.int32)

    def group(g, carry):
        wcnt, rem = carry
        pltpu.sync_copy(src_hbm.at[pl.ds(_al(g * G), G)], sbuf)
        pltpu.sync_copy(dst_hbm.at[pl.ds(_al(g * G), G)], dbuf)

        def step(t, cnt):
            sl = pl.ds(16 * t, 16)
            d16 = dbuf[sl]
            s16 = sbuf[sl]
            m = (d16 >= start) & (d16 < start + R)
            ps = plsc.cumsum(m.astype(jnp.int32))
            idx = cnt + ps - 1
            plsc.store_scatter(stg_s, [idx], s16, mask=m)
            plsc.store_scatter(stg_d, [idx], d16 - start, mask=m)
            return cnt + ps[15]

        cnt = lax.fori_loop(0, G // 16, step, rem)
        nch = cnt // 128

        def flush(i, _):
            pltpu.sync_copy(stg_s.at[pl.ds(128 * i, 128)],
                            bsrc_hbm.at[pl.ds(_al(w * EP + wcnt + 128 * i), 128)])
            pltpu.sync_copy(stg_d.at[pl.ds(128 * i, 128)],
                            bdstl_hbm.at[pl.ds(_al(w * EP + wcnt + 128 * i), 128)])
            return 0

        lax.fori_loop(0, nch, flush, 0)
        base = 128 * nch
        for i in range(8):
            stg_s[pl.ds(16 * i, 16)] = stg_s[pl.ds(base + 16 * i, 16)]
            stg_d[pl.ds(16 * i, 16)] = stg_d[pl.ds(base + 16 * i, 16)]
        return (wcnt + base, cnt - base)

    wcnt, rem = lax.fori_loop(0, E // G, group, (0, 0))
    for i in range(8):
        stg_s[pl.ds(rem + 16 * i, 16)] = zero16
        stg_d[pl.ds(rem + 16 * i, 16)] = zero16
    pltpu.sync_copy(stg_s.at[pl.ds(0, 128)],
                    bsrc_hbm.at[pl.ds(_al(w * EP + wcnt), 128)])
    pltpu.sync_copy(stg_d.at[pl.ds(0, 128)],
                    bdstl_hbm.at[pl.ds(_al(w * EP + wcnt), 128)])
    cw = wcnt + rem
    cbuf[...] = jnp.where(lax.iota(jnp.int32, 16) == 0, cw, 0)
    pltpu.sync_copy(cbuf, cnt_hbm.at[pl.ds(_al(w * 16), 16)])


def _sc_bucket(src, dst):
    ity = jnp.int32
    f = pl.kernel(
        _bucket_body,
        out_type=[
            jax.ShapeDtypeStruct((NT * EP,), ity),
            jax.ShapeDtypeStruct((NT * EP,), ity),
            jax.ShapeDtypeStruct((NT * 16,), ity),
        ],
        mesh=_mesh(),
        compiler_params=pltpu.CompilerParams(needs_layout_passes=False),
        scratch_types=[
            pltpu.VMEM((G,), ity),
            pltpu.VMEM((G,), ity),
            pltpu.VMEM((STG,), ity),
            pltpu.VMEM((STG,), ity),
            pltpu.VMEM((16,), ity),
            pltpu.SemaphoreType.DMA,
        ],
    )
    return f(src, dst)


SCAP = 8192


def _sort_body(bsrc_hbm, bdstl_hbm, cnt_hbm, osrc_hbm, odstl_hbm,
               ssrc, sdl, tsrc, tdl, cbuf, cnt, sem):
    i32 = jnp.int32
    c = lax.axis_index("c")
    s = lax.axis_index("s")
    w = c * NS + s
    ebase = w * EP
    pltpu.sync_copy(cnt_hbm.at[pl.ds(_al(w * 16), 16)], cbuf)
    cw = cbuf[...][0]
    nch = (cw + 127) // 128
    lane0 = lax.iota(i32, 16) == 0

    @pl.when((cw > 0) & (cw <= SCAP))
    def _():
        def load(i, _):
            pltpu.sync_copy(bsrc_hbm.at[pl.ds(_al(ebase + 128 * i), 128)],
                            ssrc.at[pl.ds(pl.multiple_of(128 * i, 8), 128)])
            pltpu.sync_copy(bdstl_hbm.at[pl.ds(_al(ebase + 128 * i), 128)],
                            sdl.at[pl.ds(pl.multiple_of(128 * i, 8), 128)])
            return 0

        lax.fori_loop(0, nch, load, 0)

        def zc(r, _):
            cnt[r] = 0
            return 0

        lax.fori_loop(0, R, zc, 0)

        def count(e, _):
            dl = sdl[pl.ds(e, 16)][0]
            cnt[dl] = cnt[dl] + 1
            return 0

        lax.fori_loop(0, cw, count, 0)

        def scan(r, run):
            t = cnt[r]
            cnt[r] = run
            return run + t

        lax.fori_loop(0, R, scan, 0)

        def scatter(e, _):
            dl = sdl[pl.ds(e, 16)][0]
            sv = ssrc[pl.ds(e, 16)][0]
            pos = cnt[dl]
            cnt[dl] = pos + 1
            pidx = jnp.full((16,), pos, i32)
            plsc.store_scatter(tsrc, [pidx], jnp.full((16,), sv, i32),
                               mask=lane0)
            plsc.store_scatter(tdl, [pidx], jnp.full((16,), dl, i32),
                               mask=lane0)
            return 0

        lax.fori_loop(0, cw, scatter, 0)
        zero16 = jnp.zeros((16,), i32)
        for i in range(8):
            tsrc[pl.ds(cw + 16 * i, 16)] = zero16
            tdl[pl.ds(cw + 16 * i, 16)] = zero16

        def store(i, _):
            pltpu.sync_copy(tsrc.at[pl.ds(pl.multiple_of(128 * i, 8), 128)],
                            osrc_hbm.at[pl.ds(_al(ebase + 128 * i), 128)])
            pltpu.sync_copy(tdl.at[pl.ds(pl.multiple_of(128 * i, 8), 128)],
                            odstl_hbm.at[pl.ds(_al(ebase + 128 * i), 128)])
            return 0

        lax.fori_loop(0, nch + 1, store, 0)

    @pl.when(cw > SCAP)
    def _():
        def copy(i, _):
            pltpu.sync_copy(bsrc_hbm.at[pl.ds(_al(ebase + 128 * i), 128)],
                            ssrc.at[pl.ds(0, 128)])
            pltpu.sync_copy(ssrc.at[pl.ds(0, 128)],
                            osrc_hbm.at[pl.ds(_al(ebase + 128 * i), 128)])
            pltpu.sync_copy(bdstl_hbm.at[pl.ds(_al(ebase + 128 * i), 128)],
                            sdl.at[pl.ds(0, 128)])
            pltpu.sync_copy(sdl.at[pl.ds(0, 128)],
                            odstl_hbm.at[pl.ds(_al(ebase + 128 * i), 128)])
            return 0

        lax.fori_loop(0, nch + 1, copy, 0)


def _sc_sort(bsrc, bdstl, cnts):
    ity = jnp.int32
    f = pl.kernel(
        _sort_body,
        out_type=[
            jax.ShapeDtypeStruct((NT * EP,), ity),
            jax.ShapeDtypeStruct((NT * EP,), ity),
        ],
        mesh=_mesh(),
        compiler_params=pltpu.CompilerParams(needs_layout_passes=False),
        scratch_types=[
            pltpu.VMEM((SCAP + 16,), ity),
            pltpu.VMEM((SCAP + 16,), ity),
            pltpu.VMEM((SCAP + 272,), ity),
            pltpu.VMEM((SCAP + 272,), ity),
            pltpu.VMEM((16,), ity),
            pltpu.SMEM((R,), ity),
            pltpu.SemaphoreType.DMA,
        ],
    )
    return f(bsrc, bdstl, cnts)


CH = 4096
BG = 64
NBG = CH // BG
NBLK = CH // 16


def _attn_body(q0, q1, k0, k1, v0, v1, bsrc, bdstl, cnts, xo0, xo1, scor,
               qbuf, ka, kb, esrc, edstl, escore, exb, smaxv, denomv, cbuf,
               smax, denom, sem):
    i32 = jnp.int32
    f32 = jnp.float32
    c = lax.axis_index("c")
    s = lax.axis_index("s")
    w = c * NS + s
    start = w * R
    ebase = w * EP
    pltpu.sync_copy(cnts.at[pl.ds(_al(w * 16), 16)], cbuf)
    cw = cbuf[...][0]
    nchunks = (cw + CH - 1) // CH

    def init_row(r, _):
        smax[r] = jnp.float32(-3.0e38)
        denom[r] = jnp.float32(0.0)
        return 0

    lax.fori_loop(0, R, init_row, 0)

    def passA(ch, _):
        base = _al(ebase + ch * CH)
        rem = cw - ch * CH
        nbg = jnp.minimum((rem + BG - 1) // BG, NBG)
        pltpu.sync_copy(bsrc.at[pl.ds(base, CH)], esrc)
        pltpu.sync_copy(bdstl.at[pl.ds(base, CH)], edstl.at[pl.ds(0, CH)])
        for hh in range(2):
            qh = (q0, q1)[hh]
            kh = (k0, k1)[hh]
            pltpu.sync_copy(qh.at[pl.ds(_al(start), R)], qbuf)

            pltpu.async_copy(kh.at[esrc.at[pl.ds(0, BG)]], ka, sem)

            def sddmm_from(kbuf, boff, hh=hh):
                for t in range(BG // 16):
                    def edot(i, vec, t=t, boff=boff, kbuf=kbuf):
                        ik = 16 * t + i
                        dl = edstl[pl.ds(boff + ik, 16)][0]
                        acc = [jnp.zeros((16,), f32) for _ in range(4)]
                        for j in range(HH // 16):
                            slj = pl.ds(16 * j, 16)
                            acc[j % 4] = acc[j % 4] + qbuf[dl, slj] * kbuf[ik, slj]
                        a = (acc[0] + acc[1]) + (acc[2] + acc[3])
                        return jnp.where(lax.iota(i32, 16) == i,
                                         jnp.sum(a), vec)

                    vec = lax.fori_loop(0, 16, edot, jnp.zeros((16,), f32))
                    pos = pl.ds(boff + 16 * t, 16)
                    if hh:
                        vec = vec + escore[pos]
                    escore[pos] = vec

            def gbatch(b, _, kh=kh, hh=hh):
                boff = b * BG

                @pl.when(b + 1 < nbg)
                def _():
                    noff = boff + BG

                    @pl.when((b + 1) % 2 == 0)
                    def _():
                        pltpu.async_copy(kh.at[esrc.at[pl.ds(noff, BG)]],
                                         ka, sem)

                    @pl.when((b + 1) % 2 == 1)
                    def _():
                        pltpu.async_copy(kh.at[esrc.at[pl.ds(noff, BG)]],
                                         kb, sem)

                @pl.when(b % 2 == 0)
                def _():
                    pltpu.make_async_copy(kh.at[esrc.at[pl.ds(boff, BG)]],
                                          ka, sem).wait()
                    sddmm_from(ka, boff)

                @pl.when(b % 2 == 1)
                def _():
                    pltpu.make_async_copy(kh.at[esrc.at[pl.ds(boff, BG)]],
                                          kb, sem).wait()
                    sddmm_from(kb, boff)

                return 0

            lax.fori_loop(0, nbg, gbatch, 0)

        lim = jnp.minimum(CH, rem)

        def upd(e, _):
            dl = edstl[pl.ds(e, 16)][0]
            smax[dl] = jnp.maximum(smax[dl], escore[pl.ds(e, 16)][0])
            return 0

        lax.fori_loop(0, lim, upd, 0)
        pltpu.sync_copy(escore.at[pl.ds(0, CH)], scor.at[pl.ds(base, CH)])
        return 0

    lax.fori_loop(0, nchunks, passA, 0)

    for t in range(R // 16):
        def fills(i, vs, t=t):
            r = 16 * t + i
            return jnp.where(lax.iota(i32, 16) == i, smax[r], vs)

        smaxv[pl.ds(16 * t, 16)] = lax.fori_loop(
            0, 16, fills, jnp.zeros((16,), f32))

    def passB(ch, _):
        base = _al(ebase + ch * CH)
        rem = cw - ch * CH
        pltpu.sync_copy(scor.at[pl.ds(base, CH)], escore.at[pl.ds(0, CH)])
        pltpu.sync_copy(bdstl.at[pl.ds(base, CH)], edstl.at[pl.ds(0, CH)])
        nblk = (jnp.minimum(CH, rem) + 15) // 16

        def blk(t, _):
            sl = pl.ds(16 * t, 16)
            dl16 = edstl[sl]
            eidx = lax.iota(i32, 16) + (ch * CH + 16 * t)
            dl16 = jnp.where(eidx < cw, dl16, 0)
            g = plsc.load_gather(smaxv, [dl16])
            exb[pl.ds(0, 16)] = jnp.exp(escore[sl] - g)
            limi = jnp.minimum(16, rem - 16 * t)

            def upd(i, _, t=t):
                dl = edstl[pl.ds(16 * t + i, 16)][0]
                denom[dl] = denom[dl] + exb[pl.ds(i, 16)][0]
                return 0

            lax.fori_loop(0, limi, upd, 0)
            return 0

        lax.fori_loop(0, nblk, blk, 0)
        return 0

    lax.fori_loop(0, nchunks, passB, 0)

    for t in range(R // 16):
        def filld(i, vd, t=t):
            r = 16 * t + i
            return jnp.where(lax.iota(i32, 16) == i, denom[r], vd)

        denomv[pl.ds(16 * t, 16)] = lax.fori_loop(
            0, 16, filld, jnp.zeros((16,), f32))

    for hh in range(2):
        vh = (v0, v1)[hh]

        def zrow(rr, _):
            for j in range(HH // 16):
                qbuf[rr, pl.ds(16 * j, 16)] = jnp.zeros((16,), f32)
            return 0

        lax.fori_loop(0, R, zrow, 0)

        def passC(ch, carry, hh=hh, vh=vh):
            base = _al(ebase + ch * CH)
            rem = cw - ch * CH
            npairs = jnp.minimum((rem + 2 * BG - 1) // (2 * BG), NBG // 2)
            pltpu.sync_copy(bsrc.at[pl.ds(base, CH)], esrc)
            pltpu.sync_copy(bdstl.at[pl.ds(base, CH)], edstl.at[pl.ds(0, CH)])
            pltpu.sync_copy(scor.at[pl.ds(base, CH)], escore.at[pl.ds(0, CH)])
            if hh == 0:
                def ablk(t, _):
                    sl = pl.ds(16 * t, 16)
                    dl16 = edstl[sl]
                    eidx = lax.iota(i32, 16) + (ch * CH + 16 * t)
                    valid = eidx < cw
                    dl16 = jnp.where(valid, dl16, 0)
                    g = plsc.load_gather(smaxv, [dl16])
                    ex = jnp.exp(escore[sl] - g)
                    dn = plsc.load_gather(denomv, [dl16])
                    escore[sl] = jnp.where(valid, ex / (dn + 1e-9), 0.0)
                    return 0

                lax.fori_loop(0, NBLK, ablk, 0)
                pltpu.sync_copy(escore.at[pl.ds(0, CH)],
                                scor.at[pl.ds(base, CH)])

            pltpu.async_copy(vh.at[esrc.at[pl.ds(0, BG)]], ka, sem)

            def accum_from(kbuf, boff, carry):
                def accum(i, carry, kbuf=kbuf, boff=boff):
                    cur, acc = carry
                    a = escore[pl.ds(boff + i, 16)][0]
                    dl = edstl[pl.ds(boff + i, 16)][0]
                    changed = dl != cur

                    def flushed(acc, cur=cur):
                        for j in range(HH // 16):
                            slj = pl.ds(16 * j, 16)
                            qbuf[cur, slj] = qbuf[cur, slj] + acc[j]
                        return tuple(jnp.zeros((16,), f32)
                                     for _ in range(HH // 16))

                    acc = lax.cond(changed, flushed, lambda a_: a_, acc)
                    newacc = []
                    for j in range(HH // 16):
                        slj = pl.ds(16 * j, 16)
                        newacc.append(acc[j] + kbuf[i, slj] * a)
                    return (dl, tuple(newacc))

                return lax.fori_loop(0, BG, accum, carry)

            def gbpair(p, carry, vh=vh):
                boff0 = 2 * p * BG
                boff1 = boff0 + BG
                pltpu.async_copy(vh.at[esrc.at[pl.ds(boff1, BG)]], kb, sem)
                pltpu.make_async_copy(vh.at[esrc.at[pl.ds(boff0, BG)]],
                                      ka, sem).wait()
                carry = accum_from(ka, boff0, carry)

                @pl.when(p + 1 < npairs)
                def _():
                    pltpu.async_copy(
                        vh.at[esrc.at[pl.ds(boff1 + BG, BG)]], ka, sem)

                pltpu.make_async_copy(vh.at[esrc.at[pl.ds(boff1, BG)]],
                                      kb, sem).wait()
                carry = accum_from(kb, boff1, carry)
                return carry

            return lax.fori_loop(0, npairs, gbpair, carry)

        zacc = tuple(jnp.zeros((16,), f32) for _ in range(HH // 16))
        cur, acc = lax.fori_loop(0, nchunks, passC, (0, zacc))
        for j in range(HH // 16):
            slj = pl.ds(16 * j, 16)
            qbuf[cur, slj] = qbuf[cur, slj] + acc[j]
        xoh = (xo0, xo1)[hh]
        pltpu.sync_copy(qbuf, xoh.at[pl.ds(_al(start), R)])


def _sc_attn(q0, q1, k0, k1, v0, v1, bsrc, bdstl, cnts):
    f32 = jnp.float32
    f = pl.kernel(
        _attn_body,
        out_type=[
            jax.ShapeDtypeStruct((NP, HH), f32),
            jax.ShapeDtypeStruct((NP, HH), f32),
            jax.ShapeDtypeStruct((NT * EP,), f32),
        ],
        mesh=_mesh(),
        compiler_params=pltpu.CompilerParams(needs_layout_passes=False),
        scratch_types=[
            pltpu.VMEM((R, HH), f32),
            pltpu.VMEM((BG, HH), f32),
            pltpu.VMEM((BG, HH), f32),
            pltpu.VMEM((CH,), jnp.int32),
            pltpu.VMEM((CH + 16,), jnp.int32),
            pltpu.VMEM((CH + 16,), f32),
            pltpu.VMEM((32,), f32),
            pltpu.VMEM((R,), f32),
            pltpu.VMEM((R,), f32),
            pltpu.VMEM((16,), jnp.int32),
            pltpu.SMEM((R,), f32),
            pltpu.SMEM((R,), f32),
            pltpu.SemaphoreType.DMA,
        ],
    )
    xo0, xo1, _ = f(q0, q1, k0, k1, v0, v1, bsrc, bdstl, cnts)
    return xo0, xo1


def kernel(h, in_W, in_b, qW, qb, kW, kb, vW, vb, out_W, out_b, edge_index):
    h_p = jnp.pad(h, ((0, NP - N), (0, 0)))
    x0, x1 = _tc_in_proj(h_p, in_W, in_b)
    src = edge_index[0]
    dst = edge_index[1]
    bsrc, bdstl, cnts = _sc_bucket(src, dst)
    bsrc, bdstl = _sc_sort(bsrc, bdstl, cnts)
    for l in range(L):
        q0, q1, k0, k1, v0, v1 = _tc_qkv(
            x0, x1, qW[l], qb[l], kW[l], kb[l], vW[l], vb[l])
        x0, x1 = _sc_attn(q0, q1, k0, k1, v0, v1, bsrc, bdstl, cnts)
    W_pad = jnp.pad(out_W, ((0, NCP - NCLS), (0, 0)))
    b_pad = jnp.pad(out_b, ((0, NCP - NCLS),))
    out = _tc_out(x0, x1, W_pad, b_pad)
    return out[:N, :NCLS]
